# baseline (speedup 1.0000x reference)
import functools

import jax
import jax.numpy as jnp
from jax import lax
from jax.experimental import pallas as pl
from jax.experimental.pallas import tpu as pltpu
from jax.experimental.pallas import tpu_sc as plsc

E = 8
K = 2
CAP_FACTOR = 1.5

_NC = 2
_NS = 16
_NW = _NC * _NS


def _router_kernel(x_ref, wrt_ref, d_ref, c_ref, *, capacity, trash):
    x = x_ref[...]
    wrt = wrt_ref[...]
    T = x.shape[0]
    logits = jnp.dot(x, wrt, preferred_element_type=jnp.float32)
    m = jnp.max(logits, axis=-1, keepdims=True)
    ex = jnp.exp(logits - m)
    probs = ex / jnp.sum(ex, axis=-1, keepdims=True)

    lane = jax.lax.broadcasted_iota(jnp.int32, probs.shape, 1)
    p1 = jnp.max(probs, axis=-1, keepdims=True)
    a1 = jnp.min(jnp.where(probs == p1, lane, E), axis=-1, keepdims=True)
    oh1 = (lane == a1).astype(jnp.float32)
    probs2 = jnp.where(lane == a1, -jnp.inf, probs)
    p2 = jnp.max(probs2, axis=-1, keepdims=True)
    a2 = jnp.min(jnp.where(probs2 == p2, lane, E), axis=-1, keepdims=True)
    oh2 = (lane == a2).astype(jnp.float32)

    s = p1 + p2
    p1n = p1 / s
    p2n = p2 / s

    ohsum = oh1 + oh2
    inc = ohsum
    shift = 1
    while shift < T:
        shifted = jnp.concatenate(
            [jnp.zeros((shift, E), jnp.float32), inc[: T - shift]], axis=0)
        inc = inc + shifted
        shift *= 2
    excl = inc - ohsum

    r1 = jnp.sum(oh1 * excl, axis=-1, keepdims=True).astype(jnp.int32)
    r2 = jnp.sum(oh2 * excl, axis=-1, keepdims=True).astype(jnp.int32)

    keep1 = r1 < capacity
    keep2 = r2 < capacity

    d1 = jnp.where(keep1, a1 * capacity + r1, trash)
    d2 = jnp.where(keep2, a2 * capacity + r2, trash)
    c1 = jnp.where(keep1, p1n, 0.0)
    c2 = jnp.where(keep2, p2n, 0.0)

    d_ref[...] = jnp.concatenate([d1, d2], axis=1)
    c_ref[...] = jnp.concatenate([c1, c2], axis=1)


def _sc_scatter_body(d1_hbm, d2_hbm, c1_hbm, c2_hbm, tok_hbm, w_hbm,
                     dv_v, cv_v, tok_v, w_v, *, n_slots, n_tok):
    wid = lax.axis_index("c") * _NS + lax.axis_index("s")

    @pl.when(wid == 0)
    def _():
        zi = jnp.zeros((16,), jnp.int32)
        zf = jnp.zeros((16,), jnp.float32)

        def zero_body(i, _):
            tok_v[pl.ds(i * 16, 16)] = zi
            w_v[pl.ds(i * 16, 16)] = zf
            return 0

        lax.fori_loop(0, n_slots // 16, zero_body, 0)

        iota16 = lax.iota(jnp.int32, 16)

        def scatter_from(d_hbm, c_hbm):
            pltpu.sync_copy(d_hbm, dv_v)
            pltpu.sync_copy(c_hbm, cv_v)

            def body(i, _):
                dv = dv_v[pl.ds(i * 16, 16)]
                cv = cv_v[pl.ds(i * 16, 16)]
                tokv = iota16 + i * 16
                plsc.store_scatter(tok_v, [dv], tokv)
                plsc.store_scatter(w_v, [dv], cv)
                return 0

            lax.fori_loop(0, n_tok // 16, body, 0)

        scatter_from(d1_hbm, c1_hbm)
        scatter_from(d2_hbm, c2_hbm)

        pltpu.sync_copy(tok_v, tok_hbm)
        pltpu.sync_copy(w_v, w_hbm)


def _sc_gather_body(x_hbm, tok_hbm, xg_hbm, idx_v, rows_v, sem, *, rows_per_w):
    wid = lax.axis_index("c") * _NS + lax.axis_index("s")
    base = wid * rows_per_w
    pltpu.sync_copy(tok_hbm.at[pl.ds(base, rows_per_w)], idx_v)
    pltpu.async_copy(x_hbm.at[idx_v], rows_v, sem).wait()
    pltpu.sync_copy(rows_v, xg_hbm.at[pl.ds(base, rows_per_w)])


def _ffn_kernel(xg_ref, w1_ref, b1_ref, w2_ref, b2_ref, wc_ref, out_ref):
    e = pl.program_id(0)

    @pl.when(e < E)
    def _():
        xg = xg_ref[...]
        h = jnp.dot(xg, w1_ref[0], preferred_element_type=jnp.float32)
        h = h + b1_ref[0]
        h = 0.5 * h * (1.0 + jax.lax.erf(h * 0.7071067811865476))
        o = jnp.dot(h, w2_ref[0], preferred_element_type=jnp.float32)
        o = o + b2_ref[0]
        out_ref[...] = o * wc_ref[:, 0:1]

    @pl.when(e >= E)
    def _():
        out_ref[...] = jnp.zeros_like(out_ref)


def _sc_combine_body(yw_hbm, d1_hbm, d2_hbm, out_hbm,
                     i1_v, i2_v, r1_v, r2_v, sem, *, tok_per_w, chunk, d_dim):
    wid = lax.axis_index("c") * _NS + lax.axis_index("s")
    n_chunks = tok_per_w // chunk

    def body(ci, _):
        tb = wid * tok_per_w + ci * chunk
        pltpu.sync_copy(d1_hbm.at[pl.ds(tb, chunk)], i1_v)
        pltpu.sync_copy(d2_hbm.at[pl.ds(tb, chunk)], i2_v)
        pltpu.async_copy(yw_hbm.at[i1_v], r1_v, sem).wait()
        pltpu.async_copy(yw_hbm.at[i2_v], r2_v, sem).wait()

        def add_row(i, _):
            for j in range(d_dim // 16):
                sl = pl.ds(j * 16, 16)
                r1_v[i, sl] = r1_v[i, sl] + r2_v[i, sl]
            return 0

        lax.fori_loop(0, chunk, add_row, 0)
        pltpu.sync_copy(r1_v, out_hbm.at[pl.ds(tb, chunk)])
        return 0

    lax.fori_loop(0, n_chunks, body, 0)


@jax.jit
def _moe(x, Wr, w1, b1, w2, b2):
    B, S, D = x.shape
    T = B * S
    H = w1.shape[-1]
    O = w2.shape[-1]
    capacity = int((T / E) * CAP_FACTOR)
    n_rows = E * capacity
    n_slots = n_rows + 16
    trash = n_rows

    xt = x.reshape(T, D)

    d, c = pl.pallas_call(
        functools.partial(_router_kernel, capacity=capacity, trash=trash),
        out_shape=(
            jax.ShapeDtypeStruct((T, 2), jnp.int32),
            jax.ShapeDtypeStruct((T, 2), jnp.float32),
        ),
    )(xt, Wr.T)
    d1 = d[:, 0]
    d2 = d[:, 1]
    c1 = c[:, 0]
    c2 = c[:, 1]

    mesh = plsc.VectorSubcoreMesh(core_axis_name="c", subcore_axis_name="s")
    sc_params = pltpu.CompilerParams(needs_layout_passes=False)

    tok_idx, w_pos = pl.kernel(
        functools.partial(_sc_scatter_body, n_slots=n_slots, n_tok=T),
        mesh=mesh,
        out_type=(
            jax.ShapeDtypeStruct((n_slots,), jnp.int32),
            jax.ShapeDtypeStruct((n_slots,), jnp.float32),
        ),
        scratch_types=[
            pltpu.VMEM((T,), jnp.int32),
            pltpu.VMEM((T,), jnp.float32),
            pltpu.VMEM((n_slots,), jnp.int32),
            pltpu.VMEM((n_slots,), jnp.float32),
        ],
        compiler_params=sc_params,
    )(d1, d2, c1, c2)

    rows_per_w = n_rows // _NW
    xg = pl.kernel(
        functools.partial(_sc_gather_body, rows_per_w=rows_per_w),
        mesh=mesh,
        out_type=jax.ShapeDtypeStruct((n_rows, D), jnp.float32),
        scratch_types=[
            pltpu.VMEM((rows_per_w,), jnp.int32),
            pltpu.VMEM((rows_per_w, D), jnp.float32),
            pltpu.SemaphoreType.DMA,
        ],
        compiler_params=sc_params,
    )(xt, tok_idx)

    wcol = jnp.broadcast_to(w_pos[:n_rows, None], (n_rows, 128))
    yw = pl.pallas_call(
        _ffn_kernel,
        grid=(E + 1,),
        in_specs=[
            pl.BlockSpec((capacity, D), lambda e: (jnp.where(e >= E, 0, e), 0)),
            pl.BlockSpec((1, D, H), lambda e: (jnp.where(e >= E, 0, e), 0, 0)),
            pl.BlockSpec((1, 1, H), lambda e: (jnp.where(e >= E, 0, e), 0, 0)),
            pl.BlockSpec((1, H, O), lambda e: (jnp.where(e >= E, 0, e), 0, 0)),
            pl.BlockSpec((1, 1, O), lambda e: (jnp.where(e >= E, 0, e), 0, 0)),
            pl.BlockSpec((capacity, 128), lambda e: (jnp.where(e >= E, 0, e), 0)),
        ],
        out_specs=pl.BlockSpec((capacity, O), lambda e: (e, 0)),
        out_shape=jax.ShapeDtypeStruct((n_rows + capacity, O), jnp.float32),
        compiler_params=pltpu.CompilerParams(
            dimension_semantics=("arbitrary",),
        ),
    )(xg, w1, b1.reshape(E, 1, H), w2, b2.reshape(E, 1, O), wcol)

    tok_per_w = T // _NW
    chunk = 32
    out = pl.kernel(
        functools.partial(_sc_combine_body, tok_per_w=tok_per_w,
                          chunk=chunk, d_dim=O),
        mesh=mesh,
        out_type=jax.ShapeDtypeStruct((T, O), jnp.float32),
        scratch_types=[
            pltpu.VMEM((chunk,), jnp.int32),
            pltpu.VMEM((chunk,), jnp.int32),
            pltpu.VMEM((chunk, O), jnp.float32),
            pltpu.VMEM((chunk, O), jnp.float32),
            pltpu.SemaphoreType.DMA,
        ],
        compiler_params=sc_params,
    )(yw, d1, d2)

    return out.reshape(B, S, O)


def kernel(x, Wr, w1, b1, w2, b2):
    return _moe(x, Wr, w1, b1, w2, b2)
